# baseline (speedup 1.0000x reference)
import numpy as np
import jax
import jax.numpy as jnp
from jax.experimental import pallas as pl
from jax.experimental.pallas import tpu as pltpu

_P = 16
_C = 3
_BITS = 3
_HW = 512
_G = _HW // _P
_N = _G * _G
_NCLS = 2 ** (_C * _BITS)

_EDGES = tuple(float(v) for v in np.arange(1.0 / _BITS, 1.0, 1.0 / _BITS).astype(np.float32))


def _mpp_kernel(mask_ref, logits_ref, t_ref, p_ref, r_ref, out_ref, acc_ref):
    b = pl.program_id(0)
    nb = pl.num_programs(0)

    @pl.when(b == 0)
    def _init():
        acc_ref[0] = 0.0
        acc_ref[1] = 0.0

    t = t_ref[...]
    pmat = p_ref[...]
    rmat = r_ref[...]

    lane = jax.lax.broadcasted_iota(jnp.int32, (_N, _G), 1)
    row = jax.lax.broadcasted_iota(jnp.int32, (_N, _G), 0)
    sel = lane == (row % _G)

    label = jnp.zeros((_N, 1), dtype=jnp.int32)
    for c in range(_C):
        y = jax.lax.dot(t[c], pmat, precision=jax.lax.Precision.HIGHEST)
        z = jax.lax.dot(rmat, y, precision=jax.lax.Precision.HIGHEST)
        avg = jnp.sum(jnp.where(sel, z, 0.0), axis=1, keepdims=True)
        d = ((avg > _EDGES[0]).astype(jnp.int32)
             + (avg > _EDGES[1]).astype(jnp.int32)
             + (avg > _EDGES[2]).astype(jnp.int32))
        code = jnp.where(d == 0, 4, jnp.where(d == 1, 2, jnp.where(d == 2, 1, 0)))
        label = label + code * (1 << (_BITS * (_C - 1 - c)))

    x = logits_ref[...]
    m = jnp.max(x, axis=1, keepdims=True)
    s = jnp.sum(jnp.exp(x - m), axis=1, keepdims=True)
    lse = m + jnp.log(s)
    cls = jax.lax.broadcasted_iota(jnp.int32, (_N, _NCLS), 1)
    xsel = jnp.sum(jnp.where(cls == label, x, 0.0), axis=1, keepdims=True)
    nll = lse - xsel

    mk = mask_ref[...]
    acc_ref[0] += jnp.sum(nll * mk)
    acc_ref[1] += jnp.sum(mk)

    @pl.when(b == nb - 1)
    def _finish():
        out_ref[0, 0] = acc_ref[0] / acc_ref[1]


def kernel(predicted_patches, target, mask):
    B, N, ncls = predicted_patches.shape
    mask_f = mask.astype(jnp.float32).reshape(B, N, 1)

    w = np.arange(_HW)
    pmat = jnp.asarray((w[:, None] // _P == np.arange(_G)[None, :]) * (1.0 / _P),
                       dtype=jnp.float32)
    n_idx = np.arange(_N)
    rmat = jnp.asarray((w[None, :] // _P == n_idx[:, None] // _G) * (1.0 / _P),
                       dtype=jnp.float32)

    out = pl.pallas_call(
        _mpp_kernel,
        grid=(B,),
        in_specs=[
            pl.BlockSpec((None, N, 1), lambda b: (b, 0, 0)),
            pl.BlockSpec((None, N, ncls), lambda b: (b, 0, 0)),
            pl.BlockSpec((None, _C, _HW, _HW), lambda b: (b, 0, 0, 0)),
            pl.BlockSpec((_HW, _G), lambda b: (0, 0)),
            pl.BlockSpec((_N, _HW), lambda b: (0, 0)),
        ],
        out_specs=pl.BlockSpec(memory_space=pltpu.SMEM),
        out_shape=jax.ShapeDtypeStruct((1, 1), jnp.float32),
        scratch_shapes=[pltpu.SMEM((2,), jnp.float32)],
        compiler_params=pltpu.CompilerParams(
            dimension_semantics=("arbitrary",),
        ),
    )(mask_f, predicted_patches, target, pmat, rmat)
    return out[0, 0]


# speedup vs baseline: 3.4442x; 3.4442x over previous
import numpy as np
import jax
import jax.numpy as jnp
from jax.experimental import pallas as pl
from jax.experimental.pallas import tpu as pltpu

_P = 16
_C = 3
_BITS = 3
_HW = 512
_G = _HW // _P
_N = _G * _G
_NCLS = 2 ** (_C * _BITS)

_EDGES = tuple(float(v) for v in np.arange(1.0 / _BITS, 1.0, 1.0 / _BITS).astype(np.float32))


def _mpp_kernel(mask_ref, logits_ref, t_ref, p_ref, p2_ref, r2_ref, out_ref, acc_ref):
    b = pl.program_id(0)
    nb = pl.num_programs(0)

    @pl.when(b == 0)
    def _init():
        acc_ref[0] = 0.0
        acc_ref[1] = 0.0

    t = t_ref[...]
    pmat = p_ref[...]
    p2t = p2_ref[...]
    r2 = r2_ref[...]

    lane = jax.lax.broadcasted_iota(jnp.int32, (_N, _G), 1)
    row = jax.lax.broadcasted_iota(jnp.int32, (_N, _G), 0)
    sel = lane == (row % _G)

    label = jnp.zeros((_N, 1), dtype=jnp.int32)
    for c in range(_C):
        y = jax.lax.dot(t[c], pmat)
        a32 = jax.lax.dot(p2t, y)
        z = jax.lax.dot(r2, a32)
        avg = jnp.sum(jnp.where(sel, z, 0.0), axis=1, keepdims=True)
        d = ((avg > _EDGES[0]).astype(jnp.int32)
             + (avg > _EDGES[1]).astype(jnp.int32)
             + (avg > _EDGES[2]).astype(jnp.int32))
        code = jnp.where(d == 0, 4, jnp.where(d == 1, 2, jnp.where(d == 2, 1, 0)))
        label = label + code * (1 << (_BITS * (_C - 1 - c)))

    x = logits_ref[...]
    m = jnp.max(x, axis=1, keepdims=True)
    s = jnp.sum(jnp.exp(x - m), axis=1, keepdims=True)
    lse = m + jnp.log(s)
    cls = jax.lax.broadcasted_iota(jnp.int32, (_N, _NCLS), 1)
    xsel = jnp.sum(jnp.where(cls == label, x, 0.0), axis=1, keepdims=True)
    nll = lse - xsel

    mk = mask_ref[...]
    acc_ref[0] += jnp.sum(nll * mk)
    acc_ref[1] += jnp.sum(mk)

    @pl.when(b == nb - 1)
    def _finish():
        out_ref[0, 0] = acc_ref[0] / acc_ref[1]


def kernel(predicted_patches, target, mask):
    B, N, ncls = predicted_patches.shape
    mask_f = mask.astype(jnp.float32).reshape(B, N, 1)

    w = np.arange(_HW)
    pmat = jnp.asarray((w[:, None] // _P == np.arange(_G)[None, :]) * (1.0 / _P),
                       dtype=jnp.float32)
    p2t = jnp.asarray((np.arange(_G)[:, None] == w[None, :] // _P) * (1.0 / _P),
                      dtype=jnp.float32)
    n_idx = np.arange(_N)
    r2 = jnp.asarray((n_idx[:, None] // _G == np.arange(_G)[None, :]) * 1.0,
                     dtype=jnp.float32)

    out = pl.pallas_call(
        _mpp_kernel,
        grid=(B,),
        in_specs=[
            pl.BlockSpec((None, N, 1), lambda b: (b, 0, 0)),
            pl.BlockSpec((None, N, ncls), lambda b: (b, 0, 0)),
            pl.BlockSpec((None, _C, _HW, _HW), lambda b: (b, 0, 0, 0)),
            pl.BlockSpec((_HW, _G), lambda b: (0, 0)),
            pl.BlockSpec((_G, _HW), lambda b: (0, 0)),
            pl.BlockSpec((_N, _G), lambda b: (0, 0)),
        ],
        out_specs=pl.BlockSpec(memory_space=pltpu.SMEM),
        out_shape=jax.ShapeDtypeStruct((1, 1), jnp.float32),
        scratch_shapes=[pltpu.SMEM((2,), jnp.float32)],
        compiler_params=pltpu.CompilerParams(
            dimension_semantics=("arbitrary",),
        ),
    )(mask_f, predicted_patches, target, pmat, p2t, r2)
    return out[0, 0]
